# Initial kernel scaffold; baseline (speedup 1.0000x reference)
import jax
import jax.numpy as jnp
from jax.experimental import pallas as pl


def kernel(pred_coords, true_coords, batch_vector):
    raise NotImplementedError("write your pallas kernel here")


# speedup vs baseline: 28.5538x; 28.5538x over previous
import jax
import jax.numpy as jnp
from jax.experimental import pallas as pl

_B = 16
_N = 32768
_ROWS = _N // 128


def _huber(d):
    ad = jnp.abs(d)
    return jnp.where(ad < 0.5, 0.5 * d * d, 0.5 * (ad - 0.25))


def _loss_body(p_ref, t_ref, seg_ref, out_ref):
    px, py, pz = p_ref[0], p_ref[1], p_ref[2]
    tx, ty, tz = t_ref[0], t_ref[1], t_ref[2]
    dx, dy, dz = px - tx, py - ty, pz - tz
    hubv = _huber(dx) + _huber(dy) + _huber(dz)
    sqv = dx * dx + dy * dy + dz * dz
    z = tz
    seg = seg_ref[...]

    tot_fil = jnp.float32(0.0)
    tot_ele = jnp.float32(0.0)
    for s in range(_B):
        m = seg == s
        zmin = jnp.min(jnp.where(m, z, jnp.inf))
        zmax = jnp.max(jnp.where(m, z, -jnp.inf))
        rng = zmax - zmin
        zb = zmin + 0.405 * rng
        zt = zmax - 0.405 * rng
        mid = 0.5 * (zmin + zmax)
        half = 0.19 * (zt - zb) * 0.5
        fb = mid - half
        ft = mid + half
        fil = m & (z >= zb) & (z <= zt) & (z >= fb) & (z <= ft)
        ele = m & jnp.logical_not(fil)
        fil_sum = jnp.sum(jnp.where(fil, hubv, 0.0))
        fil_cnt = jnp.sum(jnp.where(fil, 1.0, 0.0))
        ele_sum = jnp.sum(jnp.where(ele, sqv, 0.0))
        ele_cnt = jnp.sum(jnp.where(ele, 1.0, 0.0))
        fil_mean = jnp.where(fil_cnt > 0, fil_sum / (3.0 * jnp.maximum(fil_cnt, 1.0)), 0.0)
        ele_mean = jnp.where(ele_cnt > 0, ele_sum / (3.0 * jnp.maximum(ele_cnt, 1.0)), 0.0)
        tot_fil += fil_mean
        tot_ele += ele_mean

    loss = 50.0 * (tot_fil / _B) + (tot_ele / _B)
    out_ref[...] = jnp.reshape(loss, (1, 1))


@jax.jit
def kernel(pred_coords, true_coords, batch_vector):
    p = pred_coords.T.reshape(3, _ROWS, 128)
    t = true_coords.T.reshape(3, _ROWS, 128)
    seg = batch_vector.astype(jnp.int32).reshape(_ROWS, 128)
    out = pl.pallas_call(
        _loss_body,
        out_shape=jax.ShapeDtypeStruct((1, 1), jnp.float32),
    )(p, t, seg)
    return out[0, 0]
